# Initial kernel scaffold; baseline (speedup 1.0000x reference)
import jax
import jax.numpy as jnp
from jax.experimental import pallas as pl


def kernel(x):
    raise NotImplementedError("write your pallas kernel here")


# speedup vs baseline: 27.5208x; 27.5208x over previous
import math

import jax
import jax.numpy as jnp
import numpy as np
from jax.experimental import pallas as pl
from jax.experimental.pallas import tpu as pltpu

EPS = 1e-06
S = 0.025
ALPHA = 0.98
DELTA = 2.0

CHUNK = 256


def _pcen_kernel(x_ref, l_ref, d_ref, p_ref, o_ref, m_scr):
    k = pl.program_id(1)
    xb = x_ref[0]
    first = k == 0
    vec = jnp.where(first, d_ref[...], p_ref[...])
    m_prev = jnp.where(first, xb[0:1, :], m_scr[...])
    m = jax.lax.dot_general(
        l_ref[...], xb, (((1,), (0,)), ((), ())),
        preferred_element_type=jnp.float32,
        precision=jax.lax.Precision.HIGHEST,
    ) + vec * m_prev
    m_scr[...] = m[CHUNK - 1:CHUNK, :]
    o_ref[0] = jnp.sqrt(
        xb * jnp.exp(-ALPHA * jnp.log(m + EPS)) + DELTA
    ) - np.float32(math.sqrt(DELTA))


def kernel(x):
    B, T, F = x.shape
    C = CHUNK
    n_chunks = T // C
    a = 1.0 - S
    i = np.arange(C, dtype=np.float64)
    ij = i[:, None] - i[None, :]
    L = np.where(ij >= 0, S * a ** ij, 0.0).astype(np.float32)
    d = np.broadcast_to(((1.0 - S) * a ** i).astype(np.float32)[:, None], (C, F))
    p = np.broadcast_to((a ** (i + 1)).astype(np.float32)[:, None], (C, F))

    return pl.pallas_call(
        _pcen_kernel,
        grid=(B, n_chunks),
        in_specs=[
            pl.BlockSpec((1, C, F), lambda b, t: (b, t, 0)),
            pl.BlockSpec((C, C), lambda b, t: (0, 0)),
            pl.BlockSpec((C, F), lambda b, t: (0, 0)),
            pl.BlockSpec((C, F), lambda b, t: (0, 0)),
        ],
        out_specs=pl.BlockSpec((1, C, F), lambda b, t: (b, t, 0)),
        out_shape=jax.ShapeDtypeStruct((B, T, F), jnp.float32),
        scratch_shapes=[pltpu.VMEM((1, F), jnp.float32)],
        compiler_params=pltpu.CompilerParams(
            dimension_semantics=("parallel", "arbitrary"),
        ),
    )(x, jnp.asarray(L), jnp.asarray(d), jnp.asarray(p))


# speedup vs baseline: 27.6063x; 1.0031x over previous
import math

import jax
import jax.numpy as jnp
import numpy as np
from jax.experimental import pallas as pl
from jax.experimental.pallas import tpu as pltpu

EPS = 1e-06
S = 0.025
ALPHA = 0.98
DELTA = 2.0

CHUNK = 256
LANES = 128


def _pcen_kernel(x_ref, o_ref, l_scr, d_scr, p_scr, m_scr):
    k = pl.program_id(1)
    first = k == 0

    @pl.when(first)
    def _init():
        ln_a = np.float32(math.log(1.0 - S))
        ii = jax.lax.broadcasted_iota(jnp.int32, (CHUNK, CHUNK), 0)
        jj = jax.lax.broadcasted_iota(jnp.int32, (CHUNK, CHUNK), 1)
        di = (ii - jj).astype(jnp.float32)
        l_scr[...] = jnp.where(di >= 0.0, S * jnp.exp(di * ln_a), 0.0)
        ir = jax.lax.broadcasted_iota(jnp.int32, (CHUNK, LANES), 0).astype(jnp.float32)
        d_scr[...] = (1.0 - S) * jnp.exp(ir * ln_a)
        p_scr[...] = jnp.exp((ir + 1.0) * ln_a)

    xb = x_ref[0]
    vec = jnp.where(first, d_scr[...], p_scr[...])
    m_prev = jnp.where(first, xb[0:1, :], m_scr[...])
    m = jax.lax.dot_general(
        l_scr[...], xb, (((1,), (0,)), ((), ())),
        preferred_element_type=jnp.float32,
        precision=jax.lax.Precision.HIGHEST,
    ) + vec * m_prev
    m_scr[...] = m[CHUNK - 1:CHUNK, :]
    o_ref[0] = jnp.sqrt(
        xb * jnp.exp(-ALPHA * jnp.log(m + EPS)) + DELTA
    ) - np.float32(math.sqrt(DELTA))


def kernel(x):
    B, T, F = x.shape
    C = CHUNK
    return pl.pallas_call(
        _pcen_kernel,
        grid=(B, T // C),
        in_specs=[pl.BlockSpec((1, C, F), lambda b, t: (b, t, 0))],
        out_specs=pl.BlockSpec((1, C, F), lambda b, t: (b, t, 0)),
        out_shape=jax.ShapeDtypeStruct((B, T, F), jnp.float32),
        scratch_shapes=[
            pltpu.VMEM((C, C), jnp.float32),
            pltpu.VMEM((C, F), jnp.float32),
            pltpu.VMEM((C, F), jnp.float32),
            pltpu.VMEM((1, F), jnp.float32),
        ],
        compiler_params=pltpu.CompilerParams(
            dimension_semantics=("parallel", "arbitrary"),
        ),
    )(x)


# speedup vs baseline: 61.4746x; 2.2268x over previous
import math

import jax
import jax.numpy as jnp
import numpy as np
from jax.experimental import pallas as pl
from jax.experimental.pallas import tpu as pltpu

EPS = 1e-06
S = 0.025
ALPHA = 0.98
DELTA = 2.0

CHUNK = 256
BLOCK_T = 2048
LANES = 128


def _pcen(xb, m):
    return jnp.sqrt(
        xb * jnp.exp(-ALPHA * jnp.log(m + EPS)) + DELTA
    ) - np.float32(math.sqrt(DELTA))


def _pcen_kernel(x_ref, o_ref, l_scr, d_scr, p_scr, m_scr):
    k = pl.program_id(1)
    first = k == 0
    C = CHUNK

    @pl.when(first)
    def _init():
        ln_a = np.float32(math.log(1.0 - S))
        ii = jax.lax.broadcasted_iota(jnp.int32, (C, C), 0)
        jj = jax.lax.broadcasted_iota(jnp.int32, (C, C), 1)
        di = (ii - jj).astype(jnp.float32)
        l_scr[...] = jnp.where(di >= 0.0, S * jnp.exp(di * ln_a), 0.0)
        ir = jax.lax.broadcasted_iota(jnp.int32, (C, LANES), 0).astype(jnp.float32)
        d_scr[...] = (1.0 - S) * jnp.exp(ir * ln_a)
        p_scr[...] = jnp.exp((ir + 1.0) * ln_a)

    l_mat = l_scr[...]
    p_vec = p_scr[...]

    xb = x_ref[0, 0:C, :]
    vec = jnp.where(first, d_scr[...], p_vec)
    m_prev = jnp.where(first, xb[0:1, :], m_scr[...])
    m = jax.lax.dot_general(
        l_mat, xb, (((1,), (0,)), ((), ())),
        preferred_element_type=jnp.float32,
        precision=jax.lax.Precision.HIGHEST,
    ) + vec * m_prev
    o_ref[0, 0:C, :] = _pcen(xb, m)
    m_prev = m[C - 1:C, :]

    for c in range(1, BLOCK_T // C):
        xb = x_ref[0, c * C:(c + 1) * C, :]
        m = jax.lax.dot_general(
            l_mat, xb, (((1,), (0,)), ((), ())),
            preferred_element_type=jnp.float32,
            precision=jax.lax.Precision.HIGHEST,
        ) + p_vec * m_prev
        o_ref[0, c * C:(c + 1) * C, :] = _pcen(xb, m)
        m_prev = m[C - 1:C, :]

    m_scr[...] = m_prev


def kernel(x):
    B, T, F = x.shape
    return pl.pallas_call(
        _pcen_kernel,
        grid=(B, T // BLOCK_T),
        in_specs=[pl.BlockSpec((1, BLOCK_T, F), lambda b, t: (b, t, 0))],
        out_specs=pl.BlockSpec((1, BLOCK_T, F), lambda b, t: (b, t, 0)),
        out_shape=jax.ShapeDtypeStruct((B, T, F), jnp.float32),
        scratch_shapes=[
            pltpu.VMEM((CHUNK, CHUNK), jnp.float32),
            pltpu.VMEM((CHUNK, F), jnp.float32),
            pltpu.VMEM((CHUNK, F), jnp.float32),
            pltpu.VMEM((1, F), jnp.float32),
        ],
        compiler_params=pltpu.CompilerParams(
            dimension_semantics=("parallel", "arbitrary"),
        ),
    )(x)
